# Initial kernel scaffold; baseline (speedup 1.0000x reference)
import functools

import jax
import jax.numpy as jnp
from jax import lax
from jax.experimental import pallas as pl
from jax.experimental.pallas import tpu as pltpu
from jax.experimental.pallas import tpu_sc as plsc

NC = 2
NS = 16
NW = NC * NS
LANES = 16
CHUNK = 128


def _make_deg_kernel(npad, ept):
    mesh = plsc.VectorSubcoreMesh(core_axis_name="c", subcore_axis_name="s")

    @functools.partial(
        pl.kernel,
        out_type=jax.ShapeDtypeStruct((NW, npad), jnp.float32),
        mesh=mesh,
        scratch_types=[
            pltpu.VMEM((npad,), jnp.float32),
            pltpu.VMEM((ept,), jnp.int32),
        ],
    )
    def deg_kernel(col_hbm, out_hbm, hist, cols):
        c = lax.axis_index("c")
        s = lax.axis_index("s")
        w = c * NS + s
        base = pl.multiple_of(w * ept, ept)
        pltpu.sync_copy(col_hbm.at[pl.ds(base, ept)], cols)

        zero = jnp.zeros((LANES,), jnp.float32)

        def zbody(i, _):
            hist[pl.ds(pl.multiple_of(i * LANES, LANES), LANES)] = zero
            return 0

        lax.fori_loop(0, npad // LANES, zbody, 0)

        ones = jnp.ones((LANES,), jnp.float32)

        def body(i, _):
            idx = cols[pl.ds(pl.multiple_of(i * LANES, LANES), LANES)]
            plsc.addupdate_scatter(hist, [idx], ones)
            return 0

        lax.fori_loop(0, ept // LANES, body, 0)
        pltpu.sync_copy(hist, out_hbm.at[w])

    return deg_kernel


def _make_scatter_kernel(npad, ept, d):
    mesh = plsc.VectorSubcoreMesh(core_axis_name="c", subcore_axis_name="s")
    nchunks = ept // CHUNK
    rps = npad // NS

    @functools.partial(
        pl.kernel,
        out_type=jax.ShapeDtypeStruct((NC, npad, d), jnp.float32),
        mesh=mesh,
        scratch_types=[
            pltpu.VMEM_SHARED((npad, d), jnp.float32),
            pltpu.VMEM((ept,), jnp.int32),
            pltpu.VMEM((CHUNK,), jnp.int32),
            pltpu.VMEM((CHUNK, d), jnp.float32),
            pltpu.SemaphoreType.DMA,
        ],
    )
    def scat_kernel(row_hbm, col_hbm, z_hbm, zero_hbm, out_hbm,
                    acc, rows_all, cbuf, rowsv, sem):
        c = lax.axis_index("c")
        s = lax.axis_index("s")
        ebase = pl.multiple_of((c * NS + s) * ept, ept)
        nbase = pl.multiple_of(s * rps, rps)

        pltpu.sync_copy(zero_hbm, acc.at[pl.ds(nbase, rps)])
        pltpu.sync_copy(row_hbm.at[pl.ds(ebase, ept)], rows_all)
        plsc.subcore_barrier()

        def body(j, _):
            off = pl.multiple_of(j * CHUNK, CHUNK)
            pltpu.sync_copy(col_hbm.at[pl.ds(ebase + off, CHUNK)], cbuf)
            pltpu.async_copy(
                z_hbm.at[rows_all.at[pl.ds(off, CHUNK)]], rowsv, sem
            ).wait()
            pltpu.sync_copy(rowsv, acc.at[cbuf], add=True)
            return 0

        lax.fori_loop(0, nchunks, body, 0)
        plsc.subcore_barrier()
        pltpu.sync_copy(acc.at[pl.ds(nbase, rps)],
                        out_hbm.at[c, pl.ds(nbase, rps)])

    return scat_kernel


def _tc_z1(deg_p, x, w1, npad, d_in, d_h, bm=1024):
    def body(dp_ref, x_ref, w_ref, dinv_ref, z1_ref):
        deg = jnp.sum(dp_ref[...], axis=0) + 1.0
        dinv = lax.rsqrt(deg)[:, None]
        xw = jnp.dot(x_ref[...], w_ref[...], preferred_element_type=jnp.float32)
        dinv_ref[...] = dinv
        z1_ref[...] = xw * dinv

    return pl.pallas_call(
        body,
        grid=(npad // bm,),
        in_specs=[
            pl.BlockSpec((NW, bm), lambda i: (0, i)),
            pl.BlockSpec((bm, d_in), lambda i: (i, 0)),
            pl.BlockSpec((d_in, d_h), lambda i: (0, 0)),
        ],
        out_specs=[
            pl.BlockSpec((bm, 1), lambda i: (i, 0)),
            pl.BlockSpec((bm, d_h), lambda i: (i, 0)),
        ],
        out_shape=[
            jax.ShapeDtypeStruct((npad, 1), jnp.float32),
            jax.ShapeDtypeStruct((npad, d_h), jnp.float32),
        ],
    )(deg_p, x, w1)


def _tc_layer1_combine(s1_p, z1, dinv, b1, w2, npad, d_h, d_out, bm=1024):
    def body(sp_ref, z1_ref, dinv_ref, b1_ref, w2_ref, z2_ref):
        total = sp_ref[0] + sp_ref[1] + z1_ref[...]
        dinv = dinv_ref[...]
        h = jnp.maximum(total * dinv + b1_ref[...], 0.0)
        z2_ref[...] = jnp.dot(
            h, w2_ref[...], preferred_element_type=jnp.float32) * dinv

    return pl.pallas_call(
        body,
        grid=(npad // bm,),
        in_specs=[
            pl.BlockSpec((NC, bm, d_h), lambda i: (0, i, 0)),
            pl.BlockSpec((bm, d_h), lambda i: (i, 0)),
            pl.BlockSpec((bm, 1), lambda i: (i, 0)),
            pl.BlockSpec((1, d_h), lambda i: (0, 0)),
            pl.BlockSpec((d_h, d_out), lambda i: (0, 0)),
        ],
        out_specs=pl.BlockSpec((bm, d_out), lambda i: (i, 0)),
        out_shape=jax.ShapeDtypeStruct((npad, d_out), jnp.float32),
    )(s1_p, z1, dinv, b1, w2)


def _tc_layer2_combine(s2_p, z2, dinv, b2, npad, d_out, bm=1024):
    def body(sp_ref, z2_ref, dinv_ref, b2_ref, out_ref):
        total = sp_ref[0] + sp_ref[1] + z2_ref[...]
        out_ref[...] = total * dinv_ref[...] + b2_ref[...]

    return pl.pallas_call(
        body,
        grid=(npad // bm,),
        in_specs=[
            pl.BlockSpec((NC, bm, d_out), lambda i: (0, i, 0)),
            pl.BlockSpec((bm, d_out), lambda i: (i, 0)),
            pl.BlockSpec((bm, 1), lambda i: (i, 0)),
            pl.BlockSpec((1, d_out), lambda i: (0, 0)),
        ],
        out_specs=pl.BlockSpec((bm, d_out), lambda i: (i, 0)),
        out_shape=jax.ShapeDtypeStruct((npad, d_out), jnp.float32),
    )(s2_p, z2, dinv, b2)


def kernel(x, edge_index, W1, b1, W2, b2):
    n, d_in = x.shape
    d_h = W1.shape[1]
    d_out = W2.shape[1]
    e = edge_index.shape[1]

    npad = ((n + NS * LANES - 1) // (NS * LANES)) * (NS * LANES)
    epad = ((e + NW * CHUNK - 1) // (NW * CHUNK)) * (NW * CHUNK)
    ept = epad // NW

    row = edge_index[0].astype(jnp.int32)
    col = edge_index[1].astype(jnp.int32)
    pad = epad - e
    if pad:
        prow = jnp.arange(pad, dtype=jnp.int32) % n
        pcol = n + jnp.arange(pad, dtype=jnp.int32) % (npad - n)
        row = jnp.concatenate([row, prow])
        col = jnp.concatenate([col, pcol])

    xp = jnp.concatenate(
        [x, jnp.zeros((npad - n, d_in), jnp.float32)]) if npad != n else x
    b1r = b1.reshape(1, d_h)
    b2r = b2.reshape(1, d_out)
    zero_h = jnp.zeros((npad // NS, d_h), jnp.float32)
    zero_o = jnp.zeros((npad // NS, d_out), jnp.float32)

    deg_p = _make_deg_kernel(npad, ept)(col)
    dinv, z1 = _tc_z1(deg_p, xp, W1, npad, d_in, d_h)
    s1_p = _make_scatter_kernel(npad, ept, d_h)(row, col, z1, zero_h)
    z2 = _tc_layer1_combine(s1_p, z1, dinv, b1r, W2, npad, d_h, d_out)
    s2_p = _make_scatter_kernel(npad, ept, d_out)(row, col, z2, zero_o)
    out = _tc_layer2_combine(s2_p, z2, dinv, b2r, npad, d_out)
    return out[:n]


# speedup vs baseline: 27.9800x; 27.9800x over previous
import functools

import jax
import jax.numpy as jnp
from jax import lax
from jax.experimental import pallas as pl
from jax.experimental.pallas import tpu as pltpu
from jax.experimental.pallas import tpu_sc as plsc

NC = 2
NS = 16
NW = NC * NS
LANES = 16
CHUNK = 128


def _make_deg_kernel(npad, ept):
    mesh = plsc.VectorSubcoreMesh(core_axis_name="c", subcore_axis_name="s")

    @functools.partial(
        pl.kernel,
        out_type=jax.ShapeDtypeStruct((NW, npad), jnp.float32),
        mesh=mesh,
        scratch_types=[
            pltpu.VMEM((npad,), jnp.float32),
            pltpu.VMEM((ept,), jnp.int32),
        ],
        compiler_params=pltpu.CompilerParams(needs_layout_passes=False),
    )
    def deg_kernel(col_hbm, out_hbm, hist, cols):
        c = lax.axis_index("c")
        s = lax.axis_index("s")
        w = c * NS + s
        base = pl.multiple_of(w * ept, ept)
        pltpu.sync_copy(col_hbm.at[pl.ds(base, ept)], cols)

        zero = jnp.zeros((LANES,), jnp.float32)

        def zbody(i, _):
            hist[pl.ds(pl.multiple_of(i * LANES, LANES), LANES)] = zero
            return 0

        lax.fori_loop(0, npad // LANES, zbody, 0)

        ones = jnp.ones((LANES,), jnp.float32)

        def body(i, _):
            idx = cols[pl.ds(pl.multiple_of(i * LANES, LANES), LANES)]
            plsc.addupdate_scatter(hist, [idx], ones)
            return 0

        lax.fori_loop(0, ept // LANES, body, 0)
        pltpu.sync_copy(hist, out_hbm.at[w])

    return deg_kernel


def _make_scatter_kernel(npad, ept, d):
    mesh = plsc.VectorSubcoreMesh(core_axis_name="c", subcore_axis_name="s")
    nchunks = ept // CHUNK
    rps = npad // NS

    @functools.partial(
        pl.kernel,
        out_type=jax.ShapeDtypeStruct((NC, npad, d), jnp.float32),
        mesh=mesh,
        scratch_types=[
            pltpu.VMEM_SHARED((npad, d), jnp.float32),
            pltpu.VMEM((ept,), jnp.int32),
            pltpu.VMEM((CHUNK,), jnp.int32),
            pltpu.VMEM((CHUNK, d), jnp.float32),
            pltpu.SemaphoreType.DMA,
        ],
        compiler_params=pltpu.CompilerParams(use_tc_tiling_on_sc=False),
    )
    def scat_kernel(row_hbm, col_hbm, z_hbm, zero_hbm, out_hbm,
                    acc, rows_all, cbuf, rowsv, sem):
        c = lax.axis_index("c")
        s = lax.axis_index("s")
        ebase = pl.multiple_of((c * NS + s) * ept, ept)
        nbase = pl.multiple_of(s * rps, rps)

        pltpu.sync_copy(zero_hbm, acc.at[pl.ds(nbase, rps)])
        pltpu.sync_copy(row_hbm.at[pl.ds(ebase, ept)], rows_all)
        plsc.subcore_barrier()

        def body(j, _):
            off = pl.multiple_of(j * CHUNK, CHUNK)
            pltpu.sync_copy(col_hbm.at[pl.ds(ebase + off, CHUNK)], cbuf)
            pltpu.async_copy(
                z_hbm.at[rows_all.at[pl.ds(off, CHUNK)]], rowsv, sem
            ).wait()
            pltpu.sync_copy(rowsv, acc.at[cbuf], add=True)
            return 0

        lax.fori_loop(0, nchunks, body, 0)
        plsc.subcore_barrier()
        pltpu.sync_copy(acc.at[pl.ds(nbase, rps)],
                        out_hbm.at[c, pl.ds(nbase, rps)])

    return scat_kernel


def _tc_z1(deg_p, x, w1, npad, d_in, d_h, bm=1024):
    def body(dp_ref, x_ref, w_ref, dinv_ref, z1_ref):
        deg = jnp.sum(dp_ref[...], axis=0) + 1.0
        dinv = lax.rsqrt(deg)[:, None]
        xw = jnp.dot(x_ref[...], w_ref[...], preferred_element_type=jnp.float32)
        dinv_ref[...] = dinv
        z1_ref[...] = xw * dinv

    return pl.pallas_call(
        body,
        grid=(npad // bm,),
        in_specs=[
            pl.BlockSpec((NW, bm), lambda i: (0, i)),
            pl.BlockSpec((bm, d_in), lambda i: (i, 0)),
            pl.BlockSpec((d_in, d_h), lambda i: (0, 0)),
        ],
        out_specs=[
            pl.BlockSpec((bm, 1), lambda i: (i, 0)),
            pl.BlockSpec((bm, d_h), lambda i: (i, 0)),
        ],
        out_shape=[
            jax.ShapeDtypeStruct((npad, 1), jnp.float32),
            jax.ShapeDtypeStruct((npad, d_h), jnp.float32),
        ],
    )(deg_p, x, w1)


def _tc_layer1_combine(s1_p, z1, dinv, b1, w2, npad, d_h, d_out, bm=1024):
    def body(sp_ref, z1_ref, dinv_ref, b1_ref, w2_ref, z2_ref):
        total = sp_ref[0] + sp_ref[1] + z1_ref[...]
        dinv = dinv_ref[...]
        h = jnp.maximum(total * dinv + b1_ref[...], 0.0)
        z2_ref[...] = jnp.dot(
            h, w2_ref[...], preferred_element_type=jnp.float32) * dinv

    return pl.pallas_call(
        body,
        grid=(npad // bm,),
        in_specs=[
            pl.BlockSpec((NC, bm, d_h), lambda i: (0, i, 0)),
            pl.BlockSpec((bm, d_h), lambda i: (i, 0)),
            pl.BlockSpec((bm, 1), lambda i: (i, 0)),
            pl.BlockSpec((1, d_h), lambda i: (0, 0)),
            pl.BlockSpec((d_h, d_out), lambda i: (0, 0)),
        ],
        out_specs=pl.BlockSpec((bm, d_out), lambda i: (i, 0)),
        out_shape=jax.ShapeDtypeStruct((npad, d_out), jnp.float32),
    )(s1_p, z1, dinv, b1, w2)


def _tc_layer2_combine(s2_p, z2, dinv, b2, npad, d_out, bm=1024):
    def body(sp_ref, z2_ref, dinv_ref, b2_ref, out_ref):
        total = sp_ref[0] + sp_ref[1] + z2_ref[...]
        out_ref[...] = total * dinv_ref[...] + b2_ref[...]

    return pl.pallas_call(
        body,
        grid=(npad // bm,),
        in_specs=[
            pl.BlockSpec((NC, bm, d_out), lambda i: (0, i, 0)),
            pl.BlockSpec((bm, d_out), lambda i: (i, 0)),
            pl.BlockSpec((bm, 1), lambda i: (i, 0)),
            pl.BlockSpec((1, d_out), lambda i: (0, 0)),
        ],
        out_specs=pl.BlockSpec((bm, d_out), lambda i: (i, 0)),
        out_shape=jax.ShapeDtypeStruct((npad, d_out), jnp.float32),
    )(s2_p, z2, dinv, b2)


def kernel(x, edge_index, W1, b1, W2, b2):
    n, d_in = x.shape
    d_h = W1.shape[1]
    d_out = W2.shape[1]
    e = edge_index.shape[1]

    npad = ((n + NS * LANES - 1) // (NS * LANES)) * (NS * LANES)
    epad = ((e + NW * CHUNK - 1) // (NW * CHUNK)) * (NW * CHUNK)
    ept = epad // NW

    row = edge_index[0].astype(jnp.int32)
    col = edge_index[1].astype(jnp.int32)
    pad = epad - e
    if pad:
        prow = jnp.arange(pad, dtype=jnp.int32) % n
        pcol = n + jnp.arange(pad, dtype=jnp.int32) % (npad - n)
        row = jnp.concatenate([row, prow])
        col = jnp.concatenate([col, pcol])

    xp = jnp.concatenate(
        [x, jnp.zeros((npad - n, d_in), jnp.float32)]) if npad != n else x
    b1r = b1.reshape(1, d_h)
    b2r = b2.reshape(1, d_out)
    zero_h = jnp.zeros((npad // NS, d_h), jnp.float32)
    zero_o = jnp.zeros((npad // NS, d_out), jnp.float32)

    deg_p = _make_deg_kernel(npad, ept)(col)
    dinv, z1 = _tc_z1(deg_p, xp, W1, npad, d_in, d_h)
    s1_p = _make_scatter_kernel(npad, ept, d_h)(row, col, z1, zero_h)
    z2 = _tc_layer1_combine(s1_p, z1, dinv, b1r, W2, npad, d_h, d_out)
    s2_p = _make_scatter_kernel(npad, ept, d_out)(row, col, z2, zero_o)
    out = _tc_layer2_combine(s2_p, z2, dinv, b2r, npad, d_out)
    return out[:n]


# speedup vs baseline: 49.0164x; 1.7518x over previous
import functools

import jax
import jax.numpy as jnp
from jax import lax
from jax.experimental import pallas as pl
from jax.experimental.pallas import tpu as pltpu
from jax.experimental.pallas import tpu_sc as plsc

NC = 2
NS = 16
NW = NC * NS
LANES = 16
CHUNK = 128
NBUF = 4


def _make_deg_kernel(npad, ept):
    mesh = plsc.VectorSubcoreMesh(core_axis_name="c", subcore_axis_name="s")

    @functools.partial(
        pl.kernel,
        out_type=jax.ShapeDtypeStruct((NW, npad), jnp.float32),
        mesh=mesh,
        scratch_types=[
            pltpu.VMEM((npad,), jnp.float32),
            pltpu.VMEM((ept,), jnp.int32),
        ],
        compiler_params=pltpu.CompilerParams(needs_layout_passes=False),
    )
    def deg_kernel(col_hbm, out_hbm, hist, cols):
        c = lax.axis_index("c")
        s = lax.axis_index("s")
        w = c * NS + s
        base = pl.multiple_of(w * ept, ept)
        pltpu.sync_copy(col_hbm.at[pl.ds(base, ept)], cols)

        zero = jnp.zeros((LANES,), jnp.float32)

        def zbody(i, _):
            hist[pl.ds(pl.multiple_of(i * LANES, LANES), LANES)] = zero
            return 0

        lax.fori_loop(0, npad // LANES, zbody, 0)

        ones = jnp.ones((LANES,), jnp.float32)

        def body(i, _):
            idx = cols[pl.ds(pl.multiple_of(i * LANES, LANES), LANES)]
            plsc.addupdate_scatter(hist, [idx], ones)
            return 0

        lax.fori_loop(0, ept // LANES, body, 0)
        pltpu.sync_copy(hist, out_hbm.at[w])

    return deg_kernel


def _make_scatter_kernel(npad, ept, d):
    mesh = plsc.VectorSubcoreMesh(core_axis_name="c", subcore_axis_name="s")
    nchunks = ept // CHUNK
    rps = npad // NS
    assert nchunks % NBUF == 0

    @functools.partial(
        pl.kernel,
        out_type=jax.ShapeDtypeStruct((NC, npad, d), jnp.float32),
        mesh=mesh,
        scratch_types=[
            pltpu.VMEM_SHARED((npad, d), jnp.float32),
            pltpu.VMEM((nchunks, CHUNK), jnp.int32),
            pltpu.VMEM((nchunks, CHUNK), jnp.int32),
            pltpu.VMEM((NBUF, CHUNK, d), jnp.float32),
        ] + [pltpu.SemaphoreType.DMA] * NBUF,
        compiler_params=pltpu.CompilerParams(use_tc_tiling_on_sc=False),
    )
    def scat_kernel(row_hbm, col_hbm, z_hbm, zero_hbm, out_hbm,
                    acc, rows2d, cols2d, bufs, *sems):
        c = lax.axis_index("c")
        s = lax.axis_index("s")
        cbase = pl.multiple_of((c * NS + s) * nchunks, nchunks)
        nbase = pl.multiple_of(s * rps, rps)

        pltpu.sync_copy(zero_hbm, acc.at[pl.ds(nbase, rps)])
        pltpu.sync_copy(row_hbm.at[pl.ds(cbase, nchunks)], rows2d)
        pltpu.sync_copy(col_hbm.at[pl.ds(cbase, nchunks)], cols2d)
        plsc.subcore_barrier()

        def body(t, _):
            j = pl.multiple_of(t * NBUF, NBUF)
            handles = [
                pltpu.async_copy(z_hbm.at[rows2d.at[j + b]], bufs.at[b], sems[b])
                for b in range(NBUF)
            ]
            for b in range(NBUF):
                handles[b].wait()
                pltpu.sync_copy(bufs.at[b], acc.at[cols2d.at[j + b]], add=True)
            return 0

        lax.fori_loop(0, nchunks // NBUF, body, 0)
        plsc.subcore_barrier()
        pltpu.sync_copy(acc.at[pl.ds(nbase, rps)],
                        out_hbm.at[c, pl.ds(nbase, rps)])

    return scat_kernel


def _tc_z1(deg_p, x, w1, npad, d_in, d_h, bm=1024):
    def body(dp_ref, x_ref, w_ref, dinv_ref, z1_ref):
        deg = jnp.sum(dp_ref[...], axis=0) + 1.0
        dinv = lax.rsqrt(deg)[:, None]
        xw = jnp.dot(x_ref[...], w_ref[...], preferred_element_type=jnp.float32)
        dinv_ref[...] = dinv
        z1_ref[...] = xw * dinv

    return pl.pallas_call(
        body,
        grid=(npad // bm,),
        in_specs=[
            pl.BlockSpec((NW, bm), lambda i: (0, i)),
            pl.BlockSpec((bm, d_in), lambda i: (i, 0)),
            pl.BlockSpec((d_in, d_h), lambda i: (0, 0)),
        ],
        out_specs=[
            pl.BlockSpec((bm, 1), lambda i: (i, 0)),
            pl.BlockSpec((bm, d_h), lambda i: (i, 0)),
        ],
        out_shape=[
            jax.ShapeDtypeStruct((npad, 1), jnp.float32),
            jax.ShapeDtypeStruct((npad, d_h), jnp.float32),
        ],
    )(deg_p, x, w1)


def _tc_layer1_combine(s1_p, z1, dinv, b1, w2, npad, d_h, d_out, bm=1024):
    def body(sp_ref, z1_ref, dinv_ref, b1_ref, w2_ref, z2_ref):
        total = sp_ref[0] + sp_ref[1] + z1_ref[...]
        dinv = dinv_ref[...]
        h = jnp.maximum(total * dinv + b1_ref[...], 0.0)
        z2_ref[...] = jnp.dot(
            h, w2_ref[...], preferred_element_type=jnp.float32) * dinv

    return pl.pallas_call(
        body,
        grid=(npad // bm,),
        in_specs=[
            pl.BlockSpec((NC, bm, d_h), lambda i: (0, i, 0)),
            pl.BlockSpec((bm, d_h), lambda i: (i, 0)),
            pl.BlockSpec((bm, 1), lambda i: (i, 0)),
            pl.BlockSpec((1, d_h), lambda i: (0, 0)),
            pl.BlockSpec((d_h, d_out), lambda i: (0, 0)),
        ],
        out_specs=pl.BlockSpec((bm, d_out), lambda i: (i, 0)),
        out_shape=jax.ShapeDtypeStruct((npad, d_out), jnp.float32),
    )(s1_p, z1, dinv, b1, w2)


def _tc_layer2_combine(s2_p, z2, dinv, b2, npad, d_out, bm=1024):
    def body(sp_ref, z2_ref, dinv_ref, b2_ref, out_ref):
        total = sp_ref[0] + sp_ref[1] + z2_ref[...]
        out_ref[...] = total * dinv_ref[...] + b2_ref[...]

    return pl.pallas_call(
        body,
        grid=(npad // bm,),
        in_specs=[
            pl.BlockSpec((NC, bm, d_out), lambda i: (0, i, 0)),
            pl.BlockSpec((bm, d_out), lambda i: (i, 0)),
            pl.BlockSpec((bm, 1), lambda i: (i, 0)),
            pl.BlockSpec((1, d_out), lambda i: (0, 0)),
        ],
        out_specs=pl.BlockSpec((bm, d_out), lambda i: (i, 0)),
        out_shape=jax.ShapeDtypeStruct((npad, d_out), jnp.float32),
    )(s2_p, z2, dinv, b2)


def kernel(x, edge_index, W1, b1, W2, b2):
    n, d_in = x.shape
    d_h = W1.shape[1]
    d_out = W2.shape[1]
    e = edge_index.shape[1]

    npad = ((n + NS * LANES - 1) // (NS * LANES)) * (NS * LANES)
    egran = NW * CHUNK * NBUF
    epad = ((e + egran - 1) // egran) * egran
    ept = epad // NW

    row = edge_index[0].astype(jnp.int32)
    col = edge_index[1].astype(jnp.int32)
    pad = epad - e
    if pad:
        prow = jnp.arange(pad, dtype=jnp.int32) % n
        pcol = n + jnp.arange(pad, dtype=jnp.int32) % (npad - n)
        row = jnp.concatenate([row, prow])
        col = jnp.concatenate([col, pcol])

    xp = jnp.concatenate(
        [x, jnp.zeros((npad - n, d_in), jnp.float32)]) if npad != n else x
    b1r = b1.reshape(1, d_h)
    b2r = b2.reshape(1, d_out)
    zero_h = jnp.zeros((npad // NS, d_h), jnp.float32)
    zero_o = jnp.zeros((npad // NS, d_out), jnp.float32)

    row2 = row.reshape(epad // CHUNK, CHUNK)
    col2 = col.reshape(epad // CHUNK, CHUNK)

    deg_p = _make_deg_kernel(npad, ept)(col)
    dinv, z1 = _tc_z1(deg_p, xp, W1, npad, d_in, d_h)
    s1_p = _make_scatter_kernel(npad, ept, d_h)(row2, col2, z1, zero_h)
    z2 = _tc_layer1_combine(s1_p, z1, dinv, b1r, W2, npad, d_h, d_out)
    s2_p = _make_scatter_kernel(npad, ept, d_out)(row2, col2, z2, zero_o)
    out = _tc_layer2_combine(s2_p, z2, dinv, b2r, npad, d_out)
    return out[:n]


# speedup vs baseline: 50.9677x; 1.0398x over previous
import functools

import jax
import jax.numpy as jnp
from jax import lax
from jax.experimental import pallas as pl
from jax.experimental.pallas import tpu as pltpu
from jax.experimental.pallas import tpu_sc as plsc

NC = 2
NS = 16
NW = NC * NS
LANES = 16
CHUNK = 128
NBUF = 4


def _make_deg_kernel(npad, ept):
    mesh = plsc.VectorSubcoreMesh(core_axis_name="c", subcore_axis_name="s")

    @functools.partial(
        pl.kernel,
        out_type=jax.ShapeDtypeStruct((NW, npad), jnp.float32),
        mesh=mesh,
        scratch_types=[
            pltpu.VMEM((npad,), jnp.float32),
            pltpu.VMEM((ept,), jnp.int32),
        ],
        compiler_params=pltpu.CompilerParams(needs_layout_passes=False),
    )
    def deg_kernel(col_hbm, out_hbm, hist, cols):
        c = lax.axis_index("c")
        s = lax.axis_index("s")
        w = c * NS + s
        base = pl.multiple_of(w * ept, ept)
        pltpu.sync_copy(col_hbm.at[pl.ds(base, ept)], cols)

        zero = jnp.zeros((LANES,), jnp.float32)

        def zbody(i, _):
            hist[pl.ds(pl.multiple_of(i * LANES, LANES), LANES)] = zero
            return 0

        lax.fori_loop(0, npad // LANES, zbody, 0)

        ones = jnp.ones((LANES,), jnp.float32)

        def body(i, _):
            idx = cols[pl.ds(pl.multiple_of(i * LANES, LANES), LANES)]
            plsc.addupdate_scatter(hist, [idx], ones)
            return 0

        lax.fori_loop(0, ept // LANES, body, 0)
        pltpu.sync_copy(hist, out_hbm.at[w])

    return deg_kernel


def _make_scatter_kernel(npad, ept, d):
    mesh = plsc.VectorSubcoreMesh(core_axis_name="c", subcore_axis_name="s")
    nchunks = ept // CHUNK
    rps = npad // NS
    assert nchunks % NBUF == 0

    @functools.partial(
        pl.kernel,
        out_type=jax.ShapeDtypeStruct((NC, npad, d), jnp.float32),
        mesh=mesh,
        scratch_types=[
            pltpu.VMEM_SHARED((npad, d), jnp.float32),
            pltpu.VMEM((nchunks, CHUNK), jnp.int32),
            pltpu.VMEM((nchunks, CHUNK), jnp.int32),
            pltpu.VMEM((NBUF, CHUNK, d), jnp.float32),
        ] + [pltpu.SemaphoreType.DMA] * (2 * NBUF),
        compiler_params=pltpu.CompilerParams(use_tc_tiling_on_sc=False),
    )
    def scat_kernel(row_hbm, col_hbm, z_hbm, zero_hbm, out_hbm,
                    acc, rows2d, cols2d, bufs, *sems):
        c = lax.axis_index("c")
        s = lax.axis_index("s")
        cbase = pl.multiple_of((c * NS + s) * nchunks, nchunks)
        nbase = pl.multiple_of(s * rps, rps)

        pltpu.sync_copy(zero_hbm, acc.at[pl.ds(nbase, rps)])
        pltpu.sync_copy(row_hbm.at[pl.ds(cbase, nchunks)], rows2d)
        pltpu.sync_copy(col_hbm.at[pl.ds(cbase, nchunks)], cols2d)
        plsc.subcore_barrier()

        def body(t, _):
            j = pl.multiple_of(t * NBUF, NBUF)
            gh = [
                pltpu.async_copy(z_hbm.at[rows2d.at[j + b]], bufs.at[b], sems[b])
                for b in range(NBUF)
            ]
            sh = []
            for b in range(NBUF):
                gh[b].wait()
                sh.append(pltpu.async_copy(
                    bufs.at[b], acc.at[cols2d.at[j + b]], sems[NBUF + b],
                    add=True))
            for h in sh:
                h.wait()
            return 0

        lax.fori_loop(0, nchunks // NBUF, body, 0)
        plsc.subcore_barrier()
        pltpu.sync_copy(acc.at[pl.ds(nbase, rps)],
                        out_hbm.at[c, pl.ds(nbase, rps)])

    return scat_kernel


def _tc_z1(deg_p, x, w1, npad, d_in, d_h, bm=1024):
    def body(dp_ref, x_ref, w_ref, dinv_ref, z1_ref):
        deg = jnp.sum(dp_ref[...], axis=0) + 1.0
        dinv = lax.rsqrt(deg)[:, None]
        xw = jnp.dot(x_ref[...], w_ref[...], preferred_element_type=jnp.float32)
        dinv_ref[...] = dinv
        z1_ref[...] = xw * dinv

    return pl.pallas_call(
        body,
        grid=(npad // bm,),
        in_specs=[
            pl.BlockSpec((NW, bm), lambda i: (0, i)),
            pl.BlockSpec((bm, d_in), lambda i: (i, 0)),
            pl.BlockSpec((d_in, d_h), lambda i: (0, 0)),
        ],
        out_specs=[
            pl.BlockSpec((bm, 1), lambda i: (i, 0)),
            pl.BlockSpec((bm, d_h), lambda i: (i, 0)),
        ],
        out_shape=[
            jax.ShapeDtypeStruct((npad, 1), jnp.float32),
            jax.ShapeDtypeStruct((npad, d_h), jnp.float32),
        ],
    )(deg_p, x, w1)


def _tc_layer1_combine(s1_p, z1, dinv, b1, w2, npad, d_h, d_out, bm=1024):
    def body(sp_ref, z1_ref, dinv_ref, b1_ref, w2_ref, z2_ref):
        total = sp_ref[0] + sp_ref[1] + z1_ref[...]
        dinv = dinv_ref[...]
        h = jnp.maximum(total * dinv + b1_ref[...], 0.0)
        z2_ref[...] = jnp.dot(
            h, w2_ref[...], preferred_element_type=jnp.float32) * dinv

    return pl.pallas_call(
        body,
        grid=(npad // bm,),
        in_specs=[
            pl.BlockSpec((NC, bm, d_h), lambda i: (0, i, 0)),
            pl.BlockSpec((bm, d_h), lambda i: (i, 0)),
            pl.BlockSpec((bm, 1), lambda i: (i, 0)),
            pl.BlockSpec((1, d_h), lambda i: (0, 0)),
            pl.BlockSpec((d_h, d_out), lambda i: (0, 0)),
        ],
        out_specs=pl.BlockSpec((bm, d_out), lambda i: (i, 0)),
        out_shape=jax.ShapeDtypeStruct((npad, d_out), jnp.float32),
    )(s1_p, z1, dinv, b1, w2)


def _tc_layer2_combine(s2_p, z2, dinv, b2, npad, d_out, bm=1024):
    def body(sp_ref, z2_ref, dinv_ref, b2_ref, out_ref):
        total = sp_ref[0] + sp_ref[1] + z2_ref[...]
        out_ref[...] = total * dinv_ref[...] + b2_ref[...]

    return pl.pallas_call(
        body,
        grid=(npad // bm,),
        in_specs=[
            pl.BlockSpec((NC, bm, d_out), lambda i: (0, i, 0)),
            pl.BlockSpec((bm, d_out), lambda i: (i, 0)),
            pl.BlockSpec((bm, 1), lambda i: (i, 0)),
            pl.BlockSpec((1, d_out), lambda i: (0, 0)),
        ],
        out_specs=pl.BlockSpec((bm, d_out), lambda i: (i, 0)),
        out_shape=jax.ShapeDtypeStruct((npad, d_out), jnp.float32),
    )(s2_p, z2, dinv, b2)


def kernel(x, edge_index, W1, b1, W2, b2):
    n, d_in = x.shape
    d_h = W1.shape[1]
    d_out = W2.shape[1]
    e = edge_index.shape[1]

    npad = ((n + NS * LANES - 1) // (NS * LANES)) * (NS * LANES)
    egran = NW * CHUNK * NBUF
    epad = ((e + egran - 1) // egran) * egran
    ept = epad // NW

    row = edge_index[0].astype(jnp.int32)
    col = edge_index[1].astype(jnp.int32)
    pad = epad - e
    if pad:
        prow = jnp.arange(pad, dtype=jnp.int32) % n
        pcol = n + jnp.arange(pad, dtype=jnp.int32) % (npad - n)
        row = jnp.concatenate([row, prow])
        col = jnp.concatenate([col, pcol])

    xp = jnp.concatenate(
        [x, jnp.zeros((npad - n, d_in), jnp.float32)]) if npad != n else x
    b1r = b1.reshape(1, d_h)
    b2r = b2.reshape(1, d_out)
    zero_h = jnp.zeros((npad // NS, d_h), jnp.float32)
    zero_o = jnp.zeros((npad // NS, d_out), jnp.float32)

    row2 = row.reshape(epad // CHUNK, CHUNK)
    col2 = col.reshape(epad // CHUNK, CHUNK)

    deg_p = _make_deg_kernel(npad, ept)(col)
    dinv, z1 = _tc_z1(deg_p, xp, W1, npad, d_in, d_h)
    s1_p = _make_scatter_kernel(npad, ept, d_h)(row2, col2, z1, zero_h)
    z2 = _tc_layer1_combine(s1_p, z1, dinv, b1r, W2, npad, d_h, d_out)
    s2_p = _make_scatter_kernel(npad, ept, d_out)(row2, col2, z2, zero_o)
    out = _tc_layer2_combine(s2_p, z2, dinv, b2r, npad, d_out)
    return out[:n]


# speedup vs baseline: 55.0139x; 1.0794x over previous
import functools

import jax
import jax.numpy as jnp
from jax import lax
from jax.experimental import pallas as pl
from jax.experimental.pallas import tpu as pltpu
from jax.experimental.pallas import tpu_sc as plsc

NC = 2
NS = 16
NW = NC * NS
LANES = 16
CHUNK = 128
NBUF = 8


def _make_deg_kernel(npad, ept):
    mesh = plsc.VectorSubcoreMesh(core_axis_name="c", subcore_axis_name="s")

    @functools.partial(
        pl.kernel,
        out_type=jax.ShapeDtypeStruct((NW, npad), jnp.float32),
        mesh=mesh,
        scratch_types=[
            pltpu.VMEM((npad,), jnp.float32),
            pltpu.VMEM((ept,), jnp.int32),
        ],
        compiler_params=pltpu.CompilerParams(needs_layout_passes=False),
    )
    def deg_kernel(col_hbm, out_hbm, hist, cols):
        c = lax.axis_index("c")
        s = lax.axis_index("s")
        w = c * NS + s
        base = pl.multiple_of(w * ept, ept)
        pltpu.sync_copy(col_hbm.at[pl.ds(base, ept)], cols)

        zero = jnp.zeros((LANES,), jnp.float32)

        def zbody(i, _):
            hist[pl.ds(pl.multiple_of(i * LANES, LANES), LANES)] = zero
            return 0

        lax.fori_loop(0, npad // LANES, zbody, 0)

        ones = jnp.ones((LANES,), jnp.float32)

        def body(i, _):
            idx = cols[pl.ds(pl.multiple_of(i * LANES, LANES), LANES)]
            plsc.addupdate_scatter(hist, [idx], ones)
            return 0

        lax.fori_loop(0, ept // LANES, body, 0)
        pltpu.sync_copy(hist, out_hbm.at[w])

    return deg_kernel


def _make_scatter_kernel(npad, ept, d):
    mesh = plsc.VectorSubcoreMesh(core_axis_name="c", subcore_axis_name="s")
    nchunks = ept // CHUNK
    rps = npad // NS
    assert nchunks % NBUF == 0

    @functools.partial(
        pl.kernel,
        out_type=jax.ShapeDtypeStruct((NC, npad, d), jnp.float32),
        mesh=mesh,
        scratch_types=[
            pltpu.VMEM_SHARED((npad, d), jnp.float32),
            pltpu.VMEM((nchunks, CHUNK), jnp.int32),
            pltpu.VMEM((nchunks, CHUNK), jnp.int32),
            pltpu.VMEM((NBUF, CHUNK, d), jnp.float32),
        ] + [pltpu.SemaphoreType.DMA] * (2 * NBUF),
        compiler_params=pltpu.CompilerParams(use_tc_tiling_on_sc=False),
    )
    def scat_kernel(row_hbm, col_hbm, z_hbm, zero_hbm, out_hbm,
                    acc, rows2d, cols2d, bufs, *sems):
        c = lax.axis_index("c")
        s = lax.axis_index("s")
        cbase = pl.multiple_of((c * NS + s) * nchunks, nchunks)
        nbase = pl.multiple_of(s * rps, rps)

        pltpu.sync_copy(zero_hbm, acc.at[pl.ds(nbase, rps)])
        pltpu.sync_copy(row_hbm.at[pl.ds(cbase, nchunks)], rows2d)
        pltpu.sync_copy(col_hbm.at[pl.ds(cbase, nchunks)], cols2d)
        plsc.subcore_barrier()

        def body(t, _):
            j = pl.multiple_of(t * NBUF, NBUF)
            gh = [
                pltpu.async_copy(z_hbm.at[rows2d.at[j + b]], bufs.at[b], sems[b])
                for b in range(NBUF)
            ]
            sh = []
            for b in range(NBUF):
                gh[b].wait()
                sh.append(pltpu.async_copy(
                    bufs.at[b], acc.at[cols2d.at[j + b]], sems[NBUF + b],
                    add=True))
            for h in sh:
                h.wait()
            return 0

        lax.fori_loop(0, nchunks // NBUF, body, 0)
        plsc.subcore_barrier()
        pltpu.sync_copy(acc.at[pl.ds(nbase, rps)],
                        out_hbm.at[c, pl.ds(nbase, rps)])

    return scat_kernel


def _tc_z1(deg_p, x, w1, npad, d_in, d_h, bm=1024):
    def body(dp_ref, x_ref, w_ref, dinv_ref, z1_ref):
        deg = jnp.sum(dp_ref[...], axis=0) + 1.0
        dinv = lax.rsqrt(deg)[:, None]
        xw = jnp.dot(x_ref[...], w_ref[...], preferred_element_type=jnp.float32)
        dinv_ref[...] = dinv
        z1_ref[...] = xw * dinv

    return pl.pallas_call(
        body,
        grid=(npad // bm,),
        in_specs=[
            pl.BlockSpec((NW, bm), lambda i: (0, i)),
            pl.BlockSpec((bm, d_in), lambda i: (i, 0)),
            pl.BlockSpec((d_in, d_h), lambda i: (0, 0)),
        ],
        out_specs=[
            pl.BlockSpec((bm, 1), lambda i: (i, 0)),
            pl.BlockSpec((bm, d_h), lambda i: (i, 0)),
        ],
        out_shape=[
            jax.ShapeDtypeStruct((npad, 1), jnp.float32),
            jax.ShapeDtypeStruct((npad, d_h), jnp.float32),
        ],
    )(deg_p, x, w1)


def _tc_layer1_combine(s1_p, z1, dinv, b1, w2, npad, d_h, d_out, bm=1024):
    def body(sp_ref, z1_ref, dinv_ref, b1_ref, w2_ref, z2_ref):
        total = sp_ref[0] + sp_ref[1] + z1_ref[...]
        dinv = dinv_ref[...]
        h = jnp.maximum(total * dinv + b1_ref[...], 0.0)
        z2_ref[...] = jnp.dot(
            h, w2_ref[...], preferred_element_type=jnp.float32) * dinv

    return pl.pallas_call(
        body,
        grid=(npad // bm,),
        in_specs=[
            pl.BlockSpec((NC, bm, d_h), lambda i: (0, i, 0)),
            pl.BlockSpec((bm, d_h), lambda i: (i, 0)),
            pl.BlockSpec((bm, 1), lambda i: (i, 0)),
            pl.BlockSpec((1, d_h), lambda i: (0, 0)),
            pl.BlockSpec((d_h, d_out), lambda i: (0, 0)),
        ],
        out_specs=pl.BlockSpec((bm, d_out), lambda i: (i, 0)),
        out_shape=jax.ShapeDtypeStruct((npad, d_out), jnp.float32),
    )(s1_p, z1, dinv, b1, w2)


def _tc_layer2_combine(s2_p, z2, dinv, b2, npad, d_out, bm=1024):
    def body(sp_ref, z2_ref, dinv_ref, b2_ref, out_ref):
        total = sp_ref[0] + sp_ref[1] + z2_ref[...]
        out_ref[...] = total * dinv_ref[...] + b2_ref[...]

    return pl.pallas_call(
        body,
        grid=(npad // bm,),
        in_specs=[
            pl.BlockSpec((NC, bm, d_out), lambda i: (0, i, 0)),
            pl.BlockSpec((bm, d_out), lambda i: (i, 0)),
            pl.BlockSpec((bm, 1), lambda i: (i, 0)),
            pl.BlockSpec((1, d_out), lambda i: (0, 0)),
        ],
        out_specs=pl.BlockSpec((bm, d_out), lambda i: (i, 0)),
        out_shape=jax.ShapeDtypeStruct((npad, d_out), jnp.float32),
    )(s2_p, z2, dinv, b2)


def kernel(x, edge_index, W1, b1, W2, b2):
    n, d_in = x.shape
    d_h = W1.shape[1]
    d_out = W2.shape[1]
    e = edge_index.shape[1]

    npad = ((n + NS * LANES - 1) // (NS * LANES)) * (NS * LANES)
    egran = NW * CHUNK * NBUF
    epad = ((e + egran - 1) // egran) * egran
    ept = epad // NW

    row = edge_index[0].astype(jnp.int32)
    col = edge_index[1].astype(jnp.int32)
    pad = epad - e
    if pad:
        prow = jnp.arange(pad, dtype=jnp.int32) % n
        pcol = n + jnp.arange(pad, dtype=jnp.int32) % (npad - n)
        row = jnp.concatenate([row, prow])
        col = jnp.concatenate([col, pcol])

    xp = jnp.concatenate(
        [x, jnp.zeros((npad - n, d_in), jnp.float32)]) if npad != n else x
    b1r = b1.reshape(1, d_h)
    b2r = b2.reshape(1, d_out)
    zero_h = jnp.zeros((npad // NS, d_h), jnp.float32)
    zero_o = jnp.zeros((npad // NS, d_out), jnp.float32)

    row2 = row.reshape(epad // CHUNK, CHUNK)
    col2 = col.reshape(epad // CHUNK, CHUNK)

    deg_p = _make_deg_kernel(npad, ept)(col)
    dinv, z1 = _tc_z1(deg_p, xp, W1, npad, d_in, d_h)
    s1_p = _make_scatter_kernel(npad, ept, d_h)(row2, col2, z1, zero_h)
    z2 = _tc_layer1_combine(s1_p, z1, dinv, b1r, W2, npad, d_h, d_out)
    s2_p = _make_scatter_kernel(npad, ept, d_out)(row2, col2, z2, zero_o)
    out = _tc_layer2_combine(s2_p, z2, dinv, b2r, npad, d_out)
    return out[:n]


# speedup vs baseline: 55.7526x; 1.0134x over previous
import functools

import numpy as np

import jax
import jax.numpy as jnp
from jax import lax
from jax.experimental import pallas as pl
from jax.experimental.pallas import tpu as pltpu
from jax.experimental.pallas import tpu_sc as plsc

NC = 2
NS = 16
NW = NC * NS
LANES = 16
CHUNK = 128
NBUF = 8


def _make_deg_kernel(npad, ept):
    mesh = plsc.VectorSubcoreMesh(core_axis_name="c", subcore_axis_name="s")

    @functools.partial(
        pl.kernel,
        out_type=jax.ShapeDtypeStruct((NW, npad), jnp.float32),
        mesh=mesh,
        scratch_types=[
            pltpu.VMEM((npad,), jnp.float32),
            pltpu.VMEM((ept,), jnp.int32),
        ],
        compiler_params=pltpu.CompilerParams(needs_layout_passes=False),
    )
    def deg_kernel(col_hbm, out_hbm, hist, cols):
        c = lax.axis_index("c")
        s = lax.axis_index("s")
        w = c * NS + s
        base = pl.multiple_of(w * ept, ept)
        pltpu.sync_copy(col_hbm.at[pl.ds(base, ept)], cols)

        zero = jnp.zeros((LANES,), jnp.float32)

        def zbody(i, _):
            hist[pl.ds(pl.multiple_of(i * LANES, LANES), LANES)] = zero
            return 0

        lax.fori_loop(0, npad // LANES, zbody, 0)

        ones = jnp.ones((LANES,), jnp.float32)

        def body(i, _):
            idx = cols[pl.ds(pl.multiple_of(i * LANES, LANES), LANES)]
            plsc.addupdate_scatter(hist, [idx], ones)
            return 0

        lax.fori_loop(0, ept // LANES, body, 0)
        pltpu.sync_copy(hist, out_hbm.at[w])

    return deg_kernel


def _make_scatter_kernel(npad, ept, d):
    mesh = plsc.VectorSubcoreMesh(core_axis_name="c", subcore_axis_name="s")
    nchunks = ept // CHUNK
    rps = npad // NS
    assert nchunks % NBUF == 0

    @functools.partial(
        pl.kernel,
        out_type=jax.ShapeDtypeStruct((NC, npad, d), jnp.float32),
        mesh=mesh,
        scratch_types=[
            pltpu.VMEM_SHARED((npad, d), jnp.float32),
            pltpu.VMEM((nchunks, CHUNK), jnp.int32),
            pltpu.VMEM((nchunks, CHUNK), jnp.int32),
            pltpu.VMEM((NBUF, CHUNK, d), jnp.float32),
        ] + [pltpu.SemaphoreType.DMA] * (2 * NBUF),
        compiler_params=pltpu.CompilerParams(use_tc_tiling_on_sc=False),
    )
    def scat_kernel(row_hbm, col_hbm, z_hbm, zero_hbm, out_hbm,
                    acc, rows2d, cols2d, bufs, *sems):
        c = lax.axis_index("c")
        s = lax.axis_index("s")
        cbase = pl.multiple_of((c * NS + s) * nchunks, nchunks)
        nbase = pl.multiple_of(s * rps, rps)

        pltpu.sync_copy(zero_hbm, acc.at[pl.ds(nbase, rps)])
        pltpu.sync_copy(row_hbm.at[pl.ds(cbase, nchunks)], rows2d)
        pltpu.sync_copy(col_hbm.at[pl.ds(cbase, nchunks)], cols2d)
        plsc.subcore_barrier()

        def body(t, _):
            j = pl.multiple_of(t * NBUF, NBUF)
            gh = [
                pltpu.async_copy(z_hbm.at[rows2d.at[j + b]], bufs.at[b], sems[b])
                for b in range(NBUF)
            ]
            sh = []
            for b in range(NBUF):
                gh[b].wait()
                sh.append(pltpu.async_copy(
                    bufs.at[b], acc.at[cols2d.at[j + b]], sems[NBUF + b],
                    add=True))
            for h in sh:
                h.wait()
            return 0

        lax.fori_loop(0, nchunks // NBUF, body, 0)
        plsc.subcore_barrier()
        pltpu.sync_copy(acc.at[pl.ds(nbase, rps)],
                        out_hbm.at[c, pl.ds(nbase, rps)])

    return scat_kernel


def _tc_z1(deg_p, x, w1, npad, d_in, d_h, bm=1024):
    def body(dp_ref, x_ref, w_ref, degs_ref, z1_ref):
        deg = jnp.sum(dp_ref[...], axis=0) + 1.0
        dinv = lax.rsqrt(deg)[:, None]
        xw = jnp.dot(x_ref[...], w_ref[...], preferred_element_type=jnp.float32)
        degs_ref[...] = deg[None, :]
        z1_ref[...] = xw * dinv

    return pl.pallas_call(
        body,
        grid=(npad // bm,),
        in_specs=[
            pl.BlockSpec((NW, bm), lambda i: (0, i)),
            pl.BlockSpec((bm, d_in), lambda i: (i, 0)),
            pl.BlockSpec((d_in, d_h), lambda i: (0, 0)),
        ],
        out_specs=[
            pl.BlockSpec((1, bm), lambda i: (0, i)),
            pl.BlockSpec((bm, d_h), lambda i: (i, 0)),
        ],
        out_shape=[
            jax.ShapeDtypeStruct((1, npad), jnp.float32),
            jax.ShapeDtypeStruct((npad, d_h), jnp.float32),
        ],
    )(deg_p, x, w1)


def _tc_layer1_combine(s1_p, z1, degs, b1, w2, npad, d_h, d_out, bm=1024):
    def body(sp_ref, z1_ref, degs_ref, b1_ref, w2_ref, z2_ref):
        total = sp_ref[0] + sp_ref[1] + z1_ref[...]
        dinv = lax.rsqrt(degs_ref[0, :])[:, None]
        h = jnp.maximum(total * dinv + b1_ref[...], 0.0)
        z2_ref[...] = jnp.dot(
            h, w2_ref[...], preferred_element_type=jnp.float32) * dinv

    return pl.pallas_call(
        body,
        grid=(npad // bm,),
        in_specs=[
            pl.BlockSpec((NC, bm, d_h), lambda i: (0, i, 0)),
            pl.BlockSpec((bm, d_h), lambda i: (i, 0)),
            pl.BlockSpec((1, bm), lambda i: (0, i)),
            pl.BlockSpec((1, d_h), lambda i: (0, 0)),
            pl.BlockSpec((d_h, d_out), lambda i: (0, 0)),
        ],
        out_specs=pl.BlockSpec((bm, d_out), lambda i: (i, 0)),
        out_shape=jax.ShapeDtypeStruct((npad, d_out), jnp.float32),
    )(s1_p, z1, degs, b1, w2)


def _tc_layer2_combine(s2_p, z2, degs, b2, npad, d_out, bm=1024):
    def body(sp_ref, z2_ref, degs_ref, b2_ref, out_ref):
        total = sp_ref[0] + sp_ref[1] + z2_ref[...]
        dinv = lax.rsqrt(degs_ref[0, :])[:, None]
        out_ref[...] = total * dinv + b2_ref[...]

    return pl.pallas_call(
        body,
        grid=(npad // bm,),
        in_specs=[
            pl.BlockSpec((NC, bm, d_out), lambda i: (0, i, 0)),
            pl.BlockSpec((bm, d_out), lambda i: (i, 0)),
            pl.BlockSpec((1, bm), lambda i: (0, i)),
            pl.BlockSpec((1, d_out), lambda i: (0, 0)),
        ],
        out_specs=pl.BlockSpec((bm, d_out), lambda i: (i, 0)),
        out_shape=jax.ShapeDtypeStruct((npad, d_out), jnp.float32),
    )(s2_p, z2, degs, b2)


def kernel(x, edge_index, W1, b1, W2, b2):
    n, d_in = x.shape
    d_h = W1.shape[1]
    d_out = W2.shape[1]
    e = edge_index.shape[1]

    npad = ((n + NS * LANES - 1) // (NS * LANES)) * (NS * LANES)
    egran = NW * CHUNK * NBUF
    epad = ((e + egran - 1) // egran) * egran
    ept = epad // NW

    row = edge_index[0].astype(jnp.int32)
    col = edge_index[1].astype(jnp.int32)
    pad = epad - e
    if pad:
        prow = jnp.asarray(np.arange(pad, dtype=np.int32) % n)
        pcol = jnp.asarray(n + np.arange(pad, dtype=np.int32) % (npad - n))
        row = jnp.concatenate([row, prow])
        col = jnp.concatenate([col, pcol])

    xp = jnp.concatenate(
        [x, jnp.zeros((npad - n, d_in), jnp.float32)]) if npad != n else x
    b1r = b1.reshape(1, d_h)
    b2r = b2.reshape(1, d_out)
    zero_h = jnp.zeros((npad // NS, d_h), jnp.float32)
    zero_o = jnp.zeros((npad // NS, d_out), jnp.float32)

    row2 = row.reshape(epad // CHUNK, CHUNK)
    col2 = col.reshape(epad // CHUNK, CHUNK)

    deg_p = _make_deg_kernel(npad, ept)(col)
    degs, z1 = _tc_z1(deg_p, xp, W1, npad, d_in, d_h)
    s1_p = _make_scatter_kernel(npad, ept, d_h)(row2, col2, z1, zero_h)
    z2 = _tc_layer1_combine(s1_p, z1, degs, b1r, W2, npad, d_h, d_out)
    s2_p = _make_scatter_kernel(npad, ept, d_out)(row2, col2, z2, zero_o)
    out = _tc_layer2_combine(s2_p, z2, degs, b2r, npad, d_out)
    return out[:n]


# speedup vs baseline: 63.7865x; 1.1441x over previous
import functools

import numpy as np

import jax
import jax.numpy as jnp
from jax import lax
from jax.experimental import pallas as pl
from jax.experimental.pallas import tpu as pltpu
from jax.experimental.pallas import tpu_sc as plsc

NC = 2
NS = 16
NW = NC * NS
LANES = 16
CHUNK = 128
NBUF = 8


def _make_deg_kernel(npad, ept):
    mesh = plsc.VectorSubcoreMesh(core_axis_name="c", subcore_axis_name="s")

    @functools.partial(
        pl.kernel,
        out_type=jax.ShapeDtypeStruct((NW, npad), jnp.float32),
        mesh=mesh,
        scratch_types=[
            pltpu.VMEM((npad,), jnp.float32),
            pltpu.VMEM((ept,), jnp.int32),
        ],
        compiler_params=pltpu.CompilerParams(needs_layout_passes=False),
    )
    def deg_kernel(col_hbm, out_hbm, hist, cols):
        c = lax.axis_index("c")
        s = lax.axis_index("s")
        w = c * NS + s
        base = pl.multiple_of(w * ept, ept)
        pltpu.sync_copy(col_hbm.at[pl.ds(base, ept)], cols)

        zero = jnp.zeros((LANES,), jnp.float32)

        def zbody(i, _):
            hist[pl.ds(pl.multiple_of(i * LANES, LANES), LANES)] = zero
            return 0

        lax.fori_loop(0, npad // LANES, zbody, 0)

        ones = jnp.ones((LANES,), jnp.float32)

        def body(i, _):
            idx = cols[pl.ds(pl.multiple_of(i * LANES, LANES), LANES)]
            plsc.addupdate_scatter(hist, [idx], ones)
            return 0

        lax.fori_loop(0, ept // LANES, body, 0)
        pltpu.sync_copy(hist, out_hbm.at[w])

    return deg_kernel


def _make_scatter_kernel(npad, ept, d):
    mesh = plsc.VectorSubcoreMesh(core_axis_name="c", subcore_axis_name="s")
    nchunks = ept // CHUNK
    rps = npad // NS
    assert nchunks % NBUF == 0

    @functools.partial(
        pl.kernel,
        out_type=jax.ShapeDtypeStruct((NC, npad, d), jnp.float32),
        mesh=mesh,
        scratch_types=[
            pltpu.VMEM_SHARED((npad, d), jnp.float32),
            pltpu.VMEM((nchunks, CHUNK), jnp.int32),
            pltpu.VMEM((nchunks, CHUNK), jnp.int32),
            pltpu.VMEM((NBUF, CHUNK, d), jnp.float32),
        ] + [pltpu.SemaphoreType.DMA] * (2 * NBUF),
        compiler_params=pltpu.CompilerParams(use_tc_tiling_on_sc=False),
    )
    def scat_kernel(row_hbm, col_hbm, z_hbm, zero_hbm, out_hbm,
                    acc, rows2d, cols2d, bufs, *sems):
        c = lax.axis_index("c")
        s = lax.axis_index("s")
        cbase = pl.multiple_of((c * NS + s) * nchunks, nchunks)
        nbase = pl.multiple_of(s * rps, rps)

        pltpu.sync_copy(zero_hbm, acc.at[pl.ds(nbase, rps)])
        pltpu.sync_copy(row_hbm.at[pl.ds(cbase, nchunks)], rows2d)
        pltpu.sync_copy(col_hbm.at[pl.ds(cbase, nchunks)], cols2d)
        plsc.subcore_barrier()

        def body(t, _):
            j = pl.multiple_of(t * NBUF, NBUF)
            gh = [
                pltpu.async_copy(z_hbm.at[rows2d.at[j + b]], bufs.at[b], sems[b])
                for b in range(NBUF)
            ]
            sh = []
            for b in range(NBUF):
                gh[b].wait()
                sh.append(pltpu.async_copy(
                    bufs.at[b], acc.at[cols2d.at[j + b]], sems[NBUF + b],
                    add=True))
            for h in sh:
                h.wait()
            return 0

        lax.fori_loop(0, nchunks // NBUF, body, 0)
        plsc.subcore_barrier()
        pltpu.sync_copy(acc.at[pl.ds(nbase, rps)],
                        out_hbm.at[c, pl.ds(nbase, rps)])

    return scat_kernel


def _tc_z1(deg_p, x, w1, npad, d_in, d_h, bm=1024):
    def body(dp_ref, x_ref, w_ref, degs_ref, z1_ref):
        deg = jnp.sum(dp_ref[...], axis=0) + 1.0
        dinv = lax.rsqrt(deg)[:, None]
        xw = jnp.dot(x_ref[...], w_ref[...], preferred_element_type=jnp.float32)
        degs_ref[...] = deg[None, :]
        z1_ref[...] = xw * dinv

    return pl.pallas_call(
        body,
        grid=(npad // bm,),
        in_specs=[
            pl.BlockSpec((NW, bm), lambda i: (0, i)),
            pl.BlockSpec((bm, d_in), lambda i: (i, 0)),
            pl.BlockSpec((d_in, d_h), lambda i: (0, 0)),
        ],
        out_specs=[
            pl.BlockSpec((1, bm), lambda i: (0, i)),
            pl.BlockSpec((bm, d_h), lambda i: (i, 0)),
        ],
        out_shape=[
            jax.ShapeDtypeStruct((1, npad), jnp.float32),
            jax.ShapeDtypeStruct((npad, d_h), jnp.float32),
        ],
    )(deg_p, x, w1)


def _tc_layer1_combine(s1_pw, z1w, dinvw, dinv64, b1w, w2bd,
                       npad, d_h, d_out, bm=1024):
    wr = bm * d_h // 128

    def body(sp_ref, z1_ref, dw_ref, d64_ref, b1_ref, w2_ref, z2_ref):
        total = sp_ref[0] + sp_ref[1] + z1_ref[...]
        h = jnp.maximum(total * dw_ref[...] + b1_ref[...], 0.0)
        z2_ref[...] = jnp.dot(
            h, w2_ref[...], preferred_element_type=jnp.float32) * d64_ref[...]

    return pl.pallas_call(
        body,
        grid=(npad // bm,),
        in_specs=[
            pl.BlockSpec((NC, wr, 128), lambda i: (0, i, 0)),
            pl.BlockSpec((wr, 128), lambda i: (i, 0)),
            pl.BlockSpec((wr, 128), lambda i: (i, 0)),
            pl.BlockSpec((wr, 4 * d_out), lambda i: (i, 0)),
            pl.BlockSpec((1, 128), lambda i: (0, 0)),
            pl.BlockSpec((128, 4 * d_out), lambda i: (0, 0)),
        ],
        out_specs=pl.BlockSpec((wr, 4 * d_out), lambda i: (i, 0)),
        out_shape=jax.ShapeDtypeStruct((npad // 4, 4 * d_out), jnp.float32),
    )(s1_pw, z1w, dinvw, dinv64, b1w, w2bd)


def _tc_layer2_combine(s2_pw, z2w, dinv16w, b2w, npad, d_out, bm=1024):
    wo = bm * d_out // 128

    def body(sp_ref, z2_ref, dw_ref, b2_ref, out_ref):
        total = sp_ref[0] + sp_ref[1] + z2_ref[...]
        out_ref[...] = total * dw_ref[...] + b2_ref[...]

    return pl.pallas_call(
        body,
        grid=(npad // bm,),
        in_specs=[
            pl.BlockSpec((NC, wo, 128), lambda i: (0, i, 0)),
            pl.BlockSpec((wo, 128), lambda i: (i, 0)),
            pl.BlockSpec((wo, 128), lambda i: (i, 0)),
            pl.BlockSpec((1, 128), lambda i: (0, 0)),
        ],
        out_specs=pl.BlockSpec((wo, 128), lambda i: (i, 0)),
        out_shape=jax.ShapeDtypeStruct((npad * d_out // 128, 128), jnp.float32),
    )(s2_pw, z2w, dinv16w, b2w)


def kernel(x, edge_index, W1, b1, W2, b2):
    n, d_in = x.shape
    d_h = W1.shape[1]
    d_out = W2.shape[1]
    e = edge_index.shape[1]

    npad = ((n + NS * LANES - 1) // (NS * LANES)) * (NS * LANES)
    egran = NW * CHUNK * NBUF
    epad = ((e + egran - 1) // egran) * egran
    ept = epad // NW

    row = edge_index[0].astype(jnp.int32)
    col = edge_index[1].astype(jnp.int32)
    pad = epad - e
    if pad:
        prow = jnp.asarray(np.arange(pad, dtype=np.int32) % n)
        pcol = jnp.asarray(n + np.arange(pad, dtype=np.int32) % (npad - n))
        row = jnp.concatenate([row, prow])
        col = jnp.concatenate([col, pcol])

    xp = jnp.concatenate(
        [x, jnp.zeros((npad - n, d_in), jnp.float32)]) if npad != n else x
    b1r = b1.reshape(1, d_h)
    b2r = b2.reshape(1, d_out)
    zero_h = jnp.zeros((npad // NS, d_h), jnp.float32)
    zero_o = jnp.zeros((npad // NS, d_out), jnp.float32)

    row2 = row.reshape(epad // CHUNK, CHUNK)
    col2 = col.reshape(epad // CHUNK, CHUNK)

    deg_p = _make_deg_kernel(npad, ept)(col)
    degs, z1 = _tc_z1(deg_p, xp, W1, npad, d_in, d_h)

    z1w = z1.reshape(npad * d_h // 128, 128)
    dinv = lax.rsqrt(degs[0])
    dinvw = jnp.repeat(dinv, d_h).reshape(npad * d_h // 128, 128)
    dinv16 = jnp.repeat(dinv, d_out)
    dinv64 = dinv16.reshape(npad // 4, 4 * d_out)
    dinv16w = dinv16.reshape(npad * d_out // 128, 128)
    w2bd = jnp.kron(jnp.eye(4, dtype=jnp.float32), W2)
    b1w = jnp.tile(b1, 4).reshape(1, 128)
    b2w = jnp.tile(b2, 128 // d_out).reshape(1, 128)

    s1_p = _make_scatter_kernel(npad, ept, d_h)(
        row2, col2, z1w.reshape(npad, d_h), zero_h)
    z2p = _tc_layer1_combine(
        s1_p.reshape(NC, npad * d_h // 128, 128), z1w, dinvw, dinv64,
        b1w, w2bd, npad, d_h, d_out)
    z2n = z2p.reshape(npad, d_out)
    s2_p = _make_scatter_kernel(npad, ept, d_out)(row2, col2, z2n, zero_o)
    outw = _tc_layer2_combine(
        s2_p.reshape(NC, npad * d_out // 128, 128),
        z2n.reshape(npad * d_out // 128, 128), dinv16w, b2w, npad, d_out)
    return outw.reshape(npad, d_out)[:n]


# speedup vs baseline: 64.6817x; 1.0140x over previous
import functools

import numpy as np

import jax
import jax.numpy as jnp
from jax import lax
from jax.experimental import pallas as pl
from jax.experimental.pallas import tpu as pltpu
from jax.experimental.pallas import tpu_sc as plsc

NC = 2
NS = 16
NW = NC * NS
LANES = 16
CHUNK = 128
NBUF = 8


def _make_deg_kernel(npad, ept):
    mesh = plsc.VectorSubcoreMesh(core_axis_name="c", subcore_axis_name="s")

    @functools.partial(
        pl.kernel,
        out_type=jax.ShapeDtypeStruct((NW, npad), jnp.float32),
        mesh=mesh,
        scratch_types=[
            pltpu.VMEM((npad,), jnp.float32),
            pltpu.VMEM((ept,), jnp.int32),
        ],
        compiler_params=pltpu.CompilerParams(needs_layout_passes=False),
    )
    def deg_kernel(col_hbm, out_hbm, hist, cols):
        c = lax.axis_index("c")
        s = lax.axis_index("s")
        w = c * NS + s
        base = pl.multiple_of(w * ept, ept)
        pltpu.sync_copy(col_hbm.at[pl.ds(base, ept)], cols)

        zero = jnp.zeros((LANES,), jnp.float32)

        def zbody(i, _):
            hist[pl.ds(pl.multiple_of(i * LANES, LANES), LANES)] = zero
            return 0

        lax.fori_loop(0, npad // LANES, zbody, 0)

        ones = jnp.ones((LANES,), jnp.float32)

        def body(i, _):
            idx = cols[pl.ds(pl.multiple_of(i * LANES, LANES), LANES)]
            plsc.addupdate_scatter(hist, [idx], ones)
            return 0

        lax.fori_loop(0, ept // LANES, body, 0)
        pltpu.sync_copy(hist, out_hbm.at[w])

    return deg_kernel


def _make_scatter_kernel(npad, ept, d):
    mesh = plsc.VectorSubcoreMesh(core_axis_name="c", subcore_axis_name="s")
    nchunks = ept // CHUNK
    rps = npad // NS
    assert nchunks % NBUF == 0

    @functools.partial(
        pl.kernel,
        out_type=jax.ShapeDtypeStruct((NC, npad, d), jnp.float32),
        mesh=mesh,
        scratch_types=[
            pltpu.VMEM_SHARED((npad, d), jnp.float32),
            pltpu.VMEM((nchunks, CHUNK), jnp.int32),
            pltpu.VMEM((nchunks, CHUNK), jnp.int32),
            pltpu.VMEM((NBUF, CHUNK, d), jnp.float32),
        ] + [pltpu.SemaphoreType.DMA] * (2 * NBUF),
        compiler_params=pltpu.CompilerParams(use_tc_tiling_on_sc=False),
    )
    def scat_kernel(row_hbm, col_hbm, z_hbm, zero_hbm, out_hbm,
                    acc, rows2d, cols2d, bufs, *sems):
        c = lax.axis_index("c")
        s = lax.axis_index("s")
        cbase = pl.multiple_of((c * NS + s) * nchunks, nchunks)
        nbase = pl.multiple_of(s * rps, rps)

        pltpu.sync_copy(zero_hbm, acc.at[pl.ds(nbase, rps)])
        pltpu.sync_copy(row_hbm.at[pl.ds(cbase, nchunks)], rows2d)
        pltpu.sync_copy(col_hbm.at[pl.ds(cbase, nchunks)], cols2d)
        plsc.subcore_barrier()

        def body(t, _):
            j = pl.multiple_of(t * NBUF, NBUF)
            gh = [
                pltpu.async_copy(z_hbm.at[rows2d.at[j + b]], bufs.at[b], sems[b])
                for b in range(NBUF)
            ]
            sh = []
            for b in range(NBUF):
                gh[b].wait()
                sh.append(pltpu.async_copy(
                    bufs.at[b], acc.at[cols2d.at[j + b]], sems[NBUF + b],
                    add=True))
            for h in sh:
                h.wait()
            return 0

        lax.fori_loop(0, nchunks // NBUF, body, 0)
        plsc.subcore_barrier()
        pltpu.sync_copy(acc.at[pl.ds(nbase, rps)],
                        out_hbm.at[c, pl.ds(nbase, rps)])

    return scat_kernel


def _tc_z1(deg_p, x, w1, npad, d_in, d_h, bm=1024):
    def body(dp_ref, x_ref, w_ref, degs_ref, z1_ref):
        deg = jnp.sum(dp_ref[...], axis=0) + 1.0
        dinv = lax.rsqrt(deg)[:, None]
        xw = jnp.dot(x_ref[...], w_ref[...], preferred_element_type=jnp.float32)
        degs_ref[...] = deg[None, :]
        z1_ref[...] = xw * dinv

    return pl.pallas_call(
        body,
        grid=(npad // bm,),
        in_specs=[
            pl.BlockSpec((NW, bm), lambda i: (0, i)),
            pl.BlockSpec((bm, d_in), lambda i: (i, 0)),
            pl.BlockSpec((d_in, d_h), lambda i: (0, 0)),
        ],
        out_specs=[
            pl.BlockSpec((1, bm), lambda i: (0, i)),
            pl.BlockSpec((bm, d_h), lambda i: (i, 0)),
        ],
        out_shape=[
            jax.ShapeDtypeStruct((1, npad), jnp.float32),
            jax.ShapeDtypeStruct((npad, d_h), jnp.float32),
        ],
    )(deg_p, x, w1)


def _tc_layer1_combine(s1_pw, z1w, dinvw, dinv64, b1w, w2bd,
                       npad, d_h, d_out, bm=1024):
    wr = bm * d_h // 128

    def body(sp_ref, z1_ref, dw_ref, d64_ref, b1_ref, w2_ref, z2_ref):
        total = sp_ref[0] + sp_ref[1] + z1_ref[...]
        h = jnp.maximum(total * dw_ref[...] + b1_ref[...], 0.0)
        z2_ref[...] = jnp.dot(
            h, w2_ref[...], preferred_element_type=jnp.float32) * d64_ref[...]

    return pl.pallas_call(
        body,
        grid=(npad // bm,),
        in_specs=[
            pl.BlockSpec((NC, wr, 128), lambda i: (0, i, 0)),
            pl.BlockSpec((wr, 128), lambda i: (i, 0)),
            pl.BlockSpec((wr, 128), lambda i: (i, 0)),
            pl.BlockSpec((wr, 4 * d_out), lambda i: (i, 0)),
            pl.BlockSpec((1, 128), lambda i: (0, 0)),
            pl.BlockSpec((128, 4 * d_out), lambda i: (0, 0)),
        ],
        out_specs=pl.BlockSpec((wr, 4 * d_out), lambda i: (i, 0)),
        out_shape=jax.ShapeDtypeStruct((npad // 4, 4 * d_out), jnp.float32),
    )(s1_pw, z1w, dinvw, dinv64, b1w, w2bd)


def _tc_layer2_combine(s2_pw, z2w, dinv16w, b2w, n, d_out, bm=1024):
    nw_rows = n * d_out // 128
    wo = 128
    grid = (nw_rows + wo - 1) // wo

    def body(sp_ref, z2_ref, dw_ref, b2_ref, out_ref):
        total = sp_ref[0] + sp_ref[1] + z2_ref[...]
        out_ref[...] = total * dw_ref[...] + b2_ref[...]

    return pl.pallas_call(
        body,
        grid=(grid,),
        in_specs=[
            pl.BlockSpec((NC, wo, 128), lambda i: (0, i, 0)),
            pl.BlockSpec((wo, 128), lambda i: (i, 0)),
            pl.BlockSpec((wo, 128), lambda i: (i, 0)),
            pl.BlockSpec((1, 128), lambda i: (0, 0)),
        ],
        out_specs=pl.BlockSpec((wo, 128), lambda i: (i, 0)),
        out_shape=jax.ShapeDtypeStruct((nw_rows, 128), jnp.float32),
    )(s2_pw, z2w, dinv16w, b2w)


def kernel(x, edge_index, W1, b1, W2, b2):
    n, d_in = x.shape
    d_h = W1.shape[1]
    d_out = W2.shape[1]
    e = edge_index.shape[1]

    npad = ((n + NS * LANES - 1) // (NS * LANES)) * (NS * LANES)
    egran = NW * CHUNK * NBUF
    epad = ((e + egran - 1) // egran) * egran
    ept = epad // NW

    row = edge_index[0].astype(jnp.int32)
    col = edge_index[1].astype(jnp.int32)
    pad = epad - e
    if pad:
        prow = jnp.asarray(np.arange(pad, dtype=np.int32) % n)
        pcol = jnp.asarray(n + np.arange(pad, dtype=np.int32) % (npad - n))
        row = jnp.concatenate([row, prow])
        col = jnp.concatenate([col, pcol])

    xp = jnp.concatenate(
        [x, jnp.zeros((npad - n, d_in), jnp.float32)]) if npad != n else x
    b1r = b1.reshape(1, d_h)
    b2r = b2.reshape(1, d_out)
    zero_h = jnp.zeros((npad // NS, d_h), jnp.float32)
    zero_o = jnp.zeros((npad // NS, d_out), jnp.float32)

    row2 = row.reshape(epad // CHUNK, CHUNK)
    col2 = col.reshape(epad // CHUNK, CHUNK)

    assert e % (NW * LANES) == 0
    deg_p = _make_deg_kernel(npad, e // NW)(edge_index[1].astype(jnp.int32))
    degs, z1 = _tc_z1(deg_p, xp, W1, npad, d_in, d_h)

    z1w = z1.reshape(npad * d_h // 128, 128)
    dinv = lax.rsqrt(degs[0])
    dinvw = jnp.repeat(dinv, d_h).reshape(npad * d_h // 128, 128)
    dinv16 = jnp.repeat(dinv, d_out)
    dinv64 = dinv16.reshape(npad // 4, 4 * d_out)
    dinv16w = dinv16.reshape(npad * d_out // 128, 128)
    w2bd = jnp.kron(jnp.eye(4, dtype=jnp.float32), W2)
    b1w = jnp.tile(b1, 4).reshape(1, 128)
    b2w = jnp.tile(b2, 128 // d_out).reshape(1, 128)

    s1_p = _make_scatter_kernel(npad, ept, d_h)(
        row2, col2, z1w.reshape(npad, d_h), zero_h)
    z2p = _tc_layer1_combine(
        s1_p.reshape(NC, npad * d_h // 128, 128), z1w, dinvw, dinv64,
        b1w, w2bd, npad, d_h, d_out)
    z2n = z2p.reshape(npad, d_out)
    s2_p = _make_scatter_kernel(npad, ept, d_out)(row2, col2, z2n, zero_o)
    outw = _tc_layer2_combine(
        s2_p.reshape(NC, npad * d_out // 128, 128),
        z2n.reshape(npad * d_out // 128, 128), dinv16w, b2w, n, d_out)
    return outw.reshape(n, d_out)


# speedup vs baseline: 68.6175x; 1.0608x over previous
import functools

import numpy as np

import jax
import jax.numpy as jnp
from jax import lax
from jax.experimental import pallas as pl
from jax.experimental.pallas import tpu as pltpu
from jax.experimental.pallas import tpu_sc as plsc

NC = 2
NS = 16
NW = NC * NS
LANES = 16
CHUNK = 128
NBUF = 8


def _make_deg_kernel(npad, ept):
    mesh = plsc.VectorSubcoreMesh(core_axis_name="c", subcore_axis_name="s")

    @functools.partial(
        pl.kernel,
        out_type=jax.ShapeDtypeStruct((NW, npad), jnp.float32),
        mesh=mesh,
        scratch_types=[
            pltpu.VMEM((npad,), jnp.float32),
            pltpu.VMEM((ept // CHUNK, CHUNK), jnp.int32),
        ],
        compiler_params=pltpu.CompilerParams(needs_layout_passes=False),
    )
    def deg_kernel(edge_hbm, out_hbm, hist, cols):
        c = lax.axis_index("c")
        s = lax.axis_index("s")
        w = c * NS + s
        base = pl.multiple_of(w * (ept // CHUNK), ept // CHUNK)
        pltpu.sync_copy(edge_hbm.at[1, pl.ds(base, ept // CHUNK)], cols)

        zero = jnp.zeros((LANES,), jnp.float32)

        def zbody(i, _):
            hist[pl.ds(pl.multiple_of(i * LANES, LANES), LANES)] = zero
            return 0

        lax.fori_loop(0, npad // LANES, zbody, 0)

        ones = jnp.ones((LANES,), jnp.float32)

        def body(r, _):
            for b in range(CHUNK // LANES):
                idx = cols[r, pl.ds(b * LANES, LANES)]
                plsc.addupdate_scatter(hist, [idx], ones)
            return 0

        lax.fori_loop(0, ept // CHUNK, body, 0)
        pltpu.sync_copy(hist, out_hbm.at[w])

    return deg_kernel


def _make_scatter_kernel(npad, ept, d):
    mesh = plsc.VectorSubcoreMesh(core_axis_name="c", subcore_axis_name="s")
    nchunks = ept // CHUNK
    rps = npad // NS
    assert nchunks % NBUF == 0

    @functools.partial(
        pl.kernel,
        out_type=jax.ShapeDtypeStruct((NC, npad, d), jnp.float32),
        mesh=mesh,
        scratch_types=[
            pltpu.VMEM_SHARED((npad, d), jnp.float32),
            pltpu.VMEM((nchunks, CHUNK), jnp.int32),
            pltpu.VMEM((nchunks, CHUNK), jnp.int32),
            pltpu.VMEM((NBUF, CHUNK, d), jnp.float32),
        ] + [pltpu.SemaphoreType.DMA] * (2 * NBUF),
        compiler_params=pltpu.CompilerParams(use_tc_tiling_on_sc=False),
    )
    def scat_kernel(edge_hbm, z_hbm, zero_hbm, out_hbm,
                    acc, rows2d, cols2d, bufs, *sems):
        c = lax.axis_index("c")
        s = lax.axis_index("s")
        cbase = pl.multiple_of((c * NS + s) * nchunks, nchunks)
        nbase = pl.multiple_of(s * rps, rps)

        pltpu.sync_copy(zero_hbm, acc.at[pl.ds(nbase, rps)])
        pltpu.sync_copy(edge_hbm.at[0, pl.ds(cbase, nchunks)], rows2d)
        pltpu.sync_copy(edge_hbm.at[1, pl.ds(cbase, nchunks)], cols2d)
        plsc.subcore_barrier()

        def body(t, _):
            j = pl.multiple_of(t * NBUF, NBUF)
            gh = [
                pltpu.async_copy(z_hbm.at[rows2d.at[j + b]], bufs.at[b], sems[b])
                for b in range(NBUF)
            ]
            sh = []
            for b in range(NBUF):
                gh[b].wait()
                sh.append(pltpu.async_copy(
                    bufs.at[b], acc.at[cols2d.at[j + b]], sems[NBUF + b],
                    add=True))
            for h in sh:
                h.wait()
            return 0

        lax.fori_loop(0, nchunks // NBUF, body, 0)
        plsc.subcore_barrier()
        pltpu.sync_copy(acc.at[pl.ds(nbase, rps)],
                        out_hbm.at[c, pl.ds(nbase, rps)])

    return scat_kernel


def _tc_z1(deg_p, x, w1, npad, d_in, d_h, bm=1024):
    def body(dp_ref, x_ref, w_ref, degs_ref, z1_ref):
        deg = jnp.sum(dp_ref[...], axis=0) + 1.0
        dinv = lax.rsqrt(deg)[:, None]
        xw = jnp.dot(x_ref[...], w_ref[...], preferred_element_type=jnp.float32)
        degs_ref[...] = deg[None, :]
        z1_ref[...] = xw * dinv

    return pl.pallas_call(
        body,
        grid=(npad // bm,),
        in_specs=[
            pl.BlockSpec((NW, bm), lambda i: (0, i)),
            pl.BlockSpec((bm, d_in), lambda i: (i, 0)),
            pl.BlockSpec((d_in, d_h), lambda i: (0, 0)),
        ],
        out_specs=[
            pl.BlockSpec((1, bm), lambda i: (0, i)),
            pl.BlockSpec((bm, d_h), lambda i: (i, 0)),
        ],
        out_shape=[
            jax.ShapeDtypeStruct((1, npad), jnp.float32),
            jax.ShapeDtypeStruct((npad, d_h), jnp.float32),
        ],
    )(deg_p, x, w1)


def _tc_layer1_combine(s1_pw, z1w, dinvw, dinv64, b1w, w2bd,
                       npad, d_h, d_out, bm=1024):
    wr = bm * d_h // 128

    def body(sp_ref, z1_ref, dw_ref, d64_ref, b1_ref, w2_ref, z2_ref):
        total = sp_ref[0] + sp_ref[1] + z1_ref[...]
        h = jnp.maximum(total * dw_ref[...] + b1_ref[...], 0.0)
        z2_ref[...] = jnp.dot(
            h, w2_ref[...], preferred_element_type=jnp.float32) * d64_ref[...]

    return pl.pallas_call(
        body,
        grid=(npad // bm,),
        in_specs=[
            pl.BlockSpec((NC, wr, 128), lambda i: (0, i, 0)),
            pl.BlockSpec((wr, 128), lambda i: (i, 0)),
            pl.BlockSpec((wr, 128), lambda i: (i, 0)),
            pl.BlockSpec((wr, 4 * d_out), lambda i: (i, 0)),
            pl.BlockSpec((1, 128), lambda i: (0, 0)),
            pl.BlockSpec((128, 4 * d_out), lambda i: (0, 0)),
        ],
        out_specs=pl.BlockSpec((wr, 4 * d_out), lambda i: (i, 0)),
        out_shape=jax.ShapeDtypeStruct((npad // 4, 4 * d_out), jnp.float32),
    )(s1_pw, z1w, dinvw, dinv64, b1w, w2bd)


def _tc_layer2_combine(s2_pw, z2w, dinv16w, b2w, n, d_out, bm=1024):
    nw_rows = n * d_out // 128
    wo = 128
    grid = (nw_rows + wo - 1) // wo

    def body(sp_ref, z2_ref, dw_ref, b2_ref, out_ref):
        total = sp_ref[0] + sp_ref[1] + z2_ref[...]
        out_ref[...] = total * dw_ref[...] + b2_ref[...]

    return pl.pallas_call(
        body,
        grid=(grid,),
        in_specs=[
            pl.BlockSpec((NC, wo, 128), lambda i: (0, i, 0)),
            pl.BlockSpec((wo, 128), lambda i: (i, 0)),
            pl.BlockSpec((wo, 128), lambda i: (i, 0)),
            pl.BlockSpec((1, 128), lambda i: (0, 0)),
        ],
        out_specs=pl.BlockSpec((wo, 128), lambda i: (i, 0)),
        out_shape=jax.ShapeDtypeStruct((nw_rows, 128), jnp.float32),
    )(s2_pw, z2w, dinv16w, b2w)


def kernel(x, edge_index, W1, b1, W2, b2):
    n, d_in = x.shape
    d_h = W1.shape[1]
    d_out = W2.shape[1]
    e = edge_index.shape[1]

    npad = ((n + NS * LANES - 1) // (NS * LANES)) * (NS * LANES)
    egran = NW * CHUNK * NBUF
    epad = ((e + egran - 1) // egran) * egran
    ept = epad // NW

    pad = epad - e
    epad_c = jnp.asarray(np.stack([
        np.arange(pad, dtype=np.int32) % n,
        n + np.arange(pad, dtype=np.int32) % (npad - n)]))
    edge3 = jnp.concatenate(
        [edge_index.astype(jnp.int32), epad_c], axis=1
    ).reshape(2, epad // CHUNK, CHUNK)

    xp = jnp.concatenate(
        [x, jnp.zeros((npad - n, d_in), jnp.float32)]) if npad != n else x
    zero_h = jnp.zeros((npad // NS, d_h), jnp.float32)
    zero_o = jnp.zeros((npad // NS, d_out), jnp.float32)

    deg_p = _make_deg_kernel(npad, ept)(edge3)
    degs, z1 = _tc_z1(deg_p, xp, W1, npad, d_in, d_h)

    z1w = z1.reshape(npad * d_h // 128, 128)
    dinv = lax.rsqrt(degs[0])
    dinvw = jnp.repeat(dinv, d_h).reshape(npad * d_h // 128, 128)
    dinv16 = jnp.repeat(dinv, d_out)
    dinv64 = dinv16.reshape(npad // 4, 4 * d_out)
    dinv16w = dinv16.reshape(npad * d_out // 128, 128)
    w2bd = jnp.kron(jnp.eye(4, dtype=jnp.float32), W2)
    b1w = jnp.tile(b1, 4).reshape(1, 128)
    b2w = jnp.tile(b2, 128 // d_out).reshape(1, 128)

    s1_p = _make_scatter_kernel(npad, ept, d_h)(
        edge3, z1w.reshape(npad, d_h), zero_h)
    z2p = _tc_layer1_combine(
        s1_p.reshape(NC, npad * d_h // 128, 128), z1w, dinvw, dinv64,
        b1w, w2bd, npad, d_h, d_out)
    z2n = z2p.reshape(npad, d_out)
    s2_p = _make_scatter_kernel(npad, ept, d_out)(edge3, z2n, zero_o)
    outw = _tc_layer2_combine(
        s2_p.reshape(NC, npad * d_out // 128, 128),
        z2n.reshape(npad * d_out // 128, 128), dinv16w, b2w, n, d_out)
    return outw.reshape(n, d_out)
